# baseline (speedup 1.0000x reference)
import jax
import jax.numpy as jnp
from jax.experimental import pallas as pl
from jax.experimental.pallas import tpu as pltpu

_ANCHORS_WH = (
    (1.3221, 1.73145),
    (3.19275, 4.00944),
    (5.05587, 8.09892),
    (9.47112, 4.84053),
    (11.2364, 10.0071),
)
_A = 5
_C = 20
_HW = 52 * 52
_BPG = 8
_LAMBDA_COORD = 5.0
_LAMBDA_NOOBJ = 0.5


def _one_image(pred_ref, tgt_ref, sums_ref, state_ref):
    T = tgt_ref[0]
    gcls = T[0:_C, :]
    gconf = T[_C:_C + 1, :]
    gxy = T[21:23, :]
    gwh = T[23:25, :]

    conf_raw = jnp.concatenate([pred_ref[0, a * 25 + 20:a * 25 + 21, :] for a in range(_A)], axis=0)
    px = jax.nn.sigmoid(
        jnp.concatenate([pred_ref[0, a * 25 + 21:a * 25 + 22, :] for a in range(_A)], axis=0))
    py = jax.nn.sigmoid(
        jnp.concatenate([pred_ref[0, a * 25 + 22:a * 25 + 23, :] for a in range(_A)], axis=0))
    pw = jnp.concatenate(
        [jnp.exp(pred_ref[0, a * 25 + 23:a * 25 + 24, :]) * _ANCHORS_WH[a][0] for a in range(_A)],
        axis=0)
    ph = jnp.concatenate(
        [jnp.exp(pred_ref[0, a * 25 + 24:a * 25 + 25, :]) * _ANCHORS_WH[a][1] for a in range(_A)],
        axis=0)
    pconf = jax.nn.sigmoid(conf_raw)

    gx = gxy[0:1, :]
    gy = gxy[1:2, :]
    gw = gwh[0:1, :]
    gh = gwh[1:2, :]

    ix_min = jnp.maximum(px - pw * 0.5, gx - gw * 0.5)
    ix_max = jnp.minimum(px + pw * 0.5, gx + gw * 0.5)
    iy_min = jnp.maximum(py - ph * 0.5, gy - gh * 0.5)
    iy_max = jnp.minimum(py + ph * 0.5, gy + gh * 0.5)
    iw = jnp.maximum(ix_max - ix_min, 0.0)
    ih = jnp.maximum(iy_max - iy_min, 0.0)
    inter = iw * ih
    area_a = pw * ph
    area_b = gw * gh
    iou = inter / (area_a + area_b - inter + 1e-10)

    aidx = jnp.argmax(iou, axis=0).astype(jnp.int32)
    a_iota = jax.lax.broadcasted_iota(jnp.int32, (_A, _HW), 0)
    onehot = a_iota == aidx[None, :]
    obj = gconf != 0.0
    objf = obj.astype(jnp.float32)

    mfb = onehot & (gconf >= 1.0)

    sq = ((px - gx) ** 2 + (py - gy) ** 2
          + (pw - gw) ** 2 + (ph - gh) ** 2)
    box_s = jnp.sum(jnp.where(mfb, sq, 0.0))
    dconf = jnp.where(onehot & obj, pconf - 1.0, pconf)
    dconf2 = dconf * dconf
    conf_s = jnp.sum(jnp.where(mfb, dconf2, 0.0))
    all_s = jnp.sum(dconf2)

    @pl.when(state_ref[0] == 0)
    def _resolve_a0():
        any_obj = jnp.max(objf) > 0.0
        j_iota = jax.lax.broadcasted_iota(jnp.int32, (1, _HW), 1)
        big = jnp.int32(2 ** 30)
        j0 = jnp.min(jnp.where(obj, j_iota, big))
        a_here = jnp.sum(jnp.where(j_iota[0] == j0, aidx, 0))
        state_ref[0] = jnp.where(any_obj, 1, 0)
        state_ref[1] = jnp.where(any_obj, a_here, 0)

    a0 = state_ref[1]

    Csel = jax.lax.switch(
        a0, [lambda a=a: pred_ref[0, a * 25:a * 25 + _C, :] for a in range(_A)])
    cmax = jnp.max(Csel, axis=0, keepdims=True)
    ez = jnp.exp(Csel - cmax)
    lz = jnp.log(jnp.sum(ez, axis=0, keepdims=True)) + cmax
    label = jnp.argmax(gcls, axis=0).astype(jnp.int32)
    k_iota = jax.lax.broadcasted_iota(jnp.int32, (_C, _HW), 0)
    picked = jnp.sum(jnp.where(k_iota == label[None, :], Csel, 0.0),
                     axis=0, keepdims=True)
    cls_s = jnp.sum(objf * (lz - picked))

    sums_ref[0] += box_s
    sums_ref[1] += conf_s
    sums_ref[2] += all_s - conf_s
    sums_ref[3] += cls_s


def _loss_step(*refs):
    pred_refs = refs[0:_BPG]
    tgt_refs = refs[_BPG:2 * _BPG]
    sums_ref, state_ref = refs[2 * _BPG], refs[2 * _BPG + 1]
    g = pl.program_id(0)

    @pl.when(g == 0)
    def _init():
        sums_ref[0] = 0.0
        sums_ref[1] = 0.0
        sums_ref[2] = 0.0
        sums_ref[3] = 0.0
        state_ref[0] = 0
        state_ref[1] = 0

    for i in range(_BPG):
        _one_image(pred_refs[i], tgt_refs[i], sums_ref, state_ref)


def kernel(prediction, target):
    bsize = prediction.shape[0]
    pred = prediction.reshape(bsize, _A * 25, _HW)
    tgt = jnp.transpose(target, (0, 2, 1))

    def _pred_spec(i):
        return pl.BlockSpec((1, _A * 25, _HW), lambda g, i=i: (_BPG * g + i, 0, 0))

    def _tgt_spec(i):
        return pl.BlockSpec((1, 25, _HW), lambda g, i=i: (_BPG * g + i, 0, 0))

    sums, _state = pl.pallas_call(
        _loss_step,
        grid=(bsize // _BPG,),
        in_specs=[_pred_spec(i) for i in range(_BPG)]
                 + [_tgt_spec(i) for i in range(_BPG)],
        out_specs=[
            pl.BlockSpec(memory_space=pltpu.SMEM),
            pl.BlockSpec(memory_space=pltpu.SMEM),
        ],
        out_shape=[
            jax.ShapeDtypeStruct((4,), jnp.float32),
            jax.ShapeDtypeStruct((2,), jnp.int32),
        ],
        compiler_params=pltpu.CompilerParams(
            dimension_semantics=("arbitrary",),
        ),
    )(*([pred] * _BPG + [tgt] * _BPG))

    inv_b = 1.0 / bsize
    box_loss = sums[0] * (_LAMBDA_COORD * inv_b)
    conf_loss = sums[1] * inv_b
    noobj_loss = sums[2] * (_LAMBDA_NOOBJ * inv_b)
    cls_loss = sums[3] * inv_b
    return (box_loss, conf_loss, noobj_loss, cls_loss)


# speedup vs baseline: 1.0062x; 1.0062x over previous
import jax
import jax.numpy as jnp
from jax.experimental import pallas as pl
from jax.experimental.pallas import tpu as pltpu

_ANCHORS_WH = (
    (1.3221, 1.73145),
    (3.19275, 4.00944),
    (5.05587, 8.09892),
    (9.47112, 4.84053),
    (11.2364, 10.0071),
)
_A = 5
_C = 20
_HW = 52 * 52
_BPG = 4
_LAMBDA_COORD = 5.0
_LAMBDA_NOOBJ = 0.5


def _one_image(pred_ref, tgt_ref, sums_ref, state_ref):
    T = tgt_ref[0]
    gcls = T[0:_C, :]
    gconf = T[_C:_C + 1, :]
    gxy = T[21:23, :]
    gwh = T[23:25, :]

    conf_raw = jnp.concatenate([pred_ref[0, a * 25 + 20:a * 25 + 21, :] for a in range(_A)], axis=0)
    px = jax.nn.sigmoid(
        jnp.concatenate([pred_ref[0, a * 25 + 21:a * 25 + 22, :] for a in range(_A)], axis=0))
    py = jax.nn.sigmoid(
        jnp.concatenate([pred_ref[0, a * 25 + 22:a * 25 + 23, :] for a in range(_A)], axis=0))
    pw = jnp.concatenate(
        [jnp.exp(pred_ref[0, a * 25 + 23:a * 25 + 24, :]) * _ANCHORS_WH[a][0] for a in range(_A)],
        axis=0)
    ph = jnp.concatenate(
        [jnp.exp(pred_ref[0, a * 25 + 24:a * 25 + 25, :]) * _ANCHORS_WH[a][1] for a in range(_A)],
        axis=0)
    pconf = jax.nn.sigmoid(conf_raw)

    gx = gxy[0:1, :]
    gy = gxy[1:2, :]
    gw = gwh[0:1, :]
    gh = gwh[1:2, :]

    ix_min = jnp.maximum(px - pw * 0.5, gx - gw * 0.5)
    ix_max = jnp.minimum(px + pw * 0.5, gx + gw * 0.5)
    iy_min = jnp.maximum(py - ph * 0.5, gy - gh * 0.5)
    iy_max = jnp.minimum(py + ph * 0.5, gy + gh * 0.5)
    iw = jnp.maximum(ix_max - ix_min, 0.0)
    ih = jnp.maximum(iy_max - iy_min, 0.0)
    inter = iw * ih
    area_a = pw * ph
    area_b = gw * gh
    iou = inter / (area_a + area_b - inter + 1e-10)

    aidx = jnp.argmax(iou, axis=0).astype(jnp.int32)
    a_iota = jax.lax.broadcasted_iota(jnp.int32, (_A, _HW), 0)
    onehot = a_iota == aidx[None, :]
    obj = gconf != 0.0
    objf = obj.astype(jnp.float32)

    mfb = onehot & (gconf >= 1.0)

    sq = ((px - gx) ** 2 + (py - gy) ** 2
          + (pw - gw) ** 2 + (ph - gh) ** 2)
    box_s = jnp.sum(jnp.where(mfb, sq, 0.0))
    dconf = jnp.where(onehot & obj, pconf - 1.0, pconf)
    dconf2 = dconf * dconf
    conf_s = jnp.sum(jnp.where(mfb, dconf2, 0.0))
    all_s = jnp.sum(dconf2)

    @pl.when(state_ref[0] == 0)
    def _resolve_a0():
        any_obj = jnp.max(objf) > 0.0
        j_iota = jax.lax.broadcasted_iota(jnp.int32, (1, _HW), 1)
        big = jnp.int32(2 ** 30)
        j0 = jnp.min(jnp.where(obj, j_iota, big))
        a_here = jnp.sum(jnp.where(j_iota[0] == j0, aidx, 0))
        state_ref[0] = jnp.where(any_obj, 1, 0)
        state_ref[1] = jnp.where(any_obj, a_here, 0)

    a0 = state_ref[1]

    Csel = jax.lax.switch(
        a0, [lambda a=a: pred_ref[0, a * 25:a * 25 + _C, :] for a in range(_A)])
    cmax = jnp.max(Csel, axis=0, keepdims=True)
    ez = jnp.exp(Csel - cmax)
    lz = jnp.log(jnp.sum(ez, axis=0, keepdims=True)) + cmax
    label = jnp.argmax(gcls, axis=0).astype(jnp.int32)
    k_iota = jax.lax.broadcasted_iota(jnp.int32, (_C, _HW), 0)
    picked = jnp.sum(jnp.where(k_iota == label[None, :], Csel, 0.0),
                     axis=0, keepdims=True)
    cls_s = jnp.sum(objf * (lz - picked))

    sums_ref[0] += box_s
    sums_ref[1] += conf_s
    sums_ref[2] += all_s - conf_s
    sums_ref[3] += cls_s


def _loss_step(*refs):
    pred_refs = refs[0:_BPG]
    tgt_refs = refs[_BPG:2 * _BPG]
    sums_ref, state_ref = refs[2 * _BPG], refs[2 * _BPG + 1]
    g = pl.program_id(0)

    @pl.when(g == 0)
    def _init():
        sums_ref[0] = 0.0
        sums_ref[1] = 0.0
        sums_ref[2] = 0.0
        sums_ref[3] = 0.0
        state_ref[0] = 0
        state_ref[1] = 0

    for i in range(_BPG):
        _one_image(pred_refs[i], tgt_refs[i], sums_ref, state_ref)


def kernel(prediction, target):
    bsize = prediction.shape[0]
    pred = prediction.reshape(bsize, _A * 25, _HW)
    tgt = jnp.transpose(target, (0, 2, 1))

    def _pred_spec(i):
        return pl.BlockSpec((1, _A * 25, _HW), lambda g, i=i: (_BPG * g + i, 0, 0))

    def _tgt_spec(i):
        return pl.BlockSpec((1, 25, _HW), lambda g, i=i: (_BPG * g + i, 0, 0))

    sums, _state = pl.pallas_call(
        _loss_step,
        grid=(bsize // _BPG,),
        in_specs=[_pred_spec(i) for i in range(_BPG)]
                 + [_tgt_spec(i) for i in range(_BPG)],
        out_specs=[
            pl.BlockSpec(memory_space=pltpu.SMEM),
            pl.BlockSpec(memory_space=pltpu.SMEM),
        ],
        out_shape=[
            jax.ShapeDtypeStruct((4,), jnp.float32),
            jax.ShapeDtypeStruct((2,), jnp.int32),
        ],
        compiler_params=pltpu.CompilerParams(
            dimension_semantics=("arbitrary",),
        ),
    )(*([pred] * _BPG + [tgt] * _BPG))

    inv_b = 1.0 / bsize
    box_loss = sums[0] * (_LAMBDA_COORD * inv_b)
    conf_loss = sums[1] * inv_b
    noobj_loss = sums[2] * (_LAMBDA_NOOBJ * inv_b)
    cls_loss = sums[3] * inv_b
    return (box_loss, conf_loss, noobj_loss, cls_loss)


# speedup vs baseline: 1.0785x; 1.0718x over previous
import jax
import jax.numpy as jnp
from jax.experimental import pallas as pl
from jax.experimental.pallas import tpu as pltpu

_ANCHORS_WH = (
    (1.3221, 1.73145),
    (3.19275, 4.00944),
    (5.05587, 8.09892),
    (9.47112, 4.84053),
    (11.2364, 10.0071),
)
_A = 5
_C = 20
_HW = 52 * 52
_BPG = 4
_LAMBDA_COORD = 5.0
_LAMBDA_NOOBJ = 0.5


def _one_image(pred_ref, tgt_ref, acc_ref, state_ref):
    T = tgt_ref[0]
    gcls = T[0:_C, :]
    gconf = T[_C:_C + 1, :]
    gxy = T[21:23, :]
    gwh = T[23:25, :]

    conf_raw = jnp.concatenate([pred_ref[0, a * 25 + 20:a * 25 + 21, :] for a in range(_A)], axis=0)
    px = jax.nn.sigmoid(
        jnp.concatenate([pred_ref[0, a * 25 + 21:a * 25 + 22, :] for a in range(_A)], axis=0))
    py = jax.nn.sigmoid(
        jnp.concatenate([pred_ref[0, a * 25 + 22:a * 25 + 23, :] for a in range(_A)], axis=0))
    pw = jnp.concatenate(
        [jnp.exp(pred_ref[0, a * 25 + 23:a * 25 + 24, :]) * _ANCHORS_WH[a][0] for a in range(_A)],
        axis=0)
    ph = jnp.concatenate(
        [jnp.exp(pred_ref[0, a * 25 + 24:a * 25 + 25, :]) * _ANCHORS_WH[a][1] for a in range(_A)],
        axis=0)
    pconf = jax.nn.sigmoid(conf_raw)

    gx = gxy[0:1, :]
    gy = gxy[1:2, :]
    gw = gwh[0:1, :]
    gh = gwh[1:2, :]

    ix_min = jnp.maximum(px - pw * 0.5, gx - gw * 0.5)
    ix_max = jnp.minimum(px + pw * 0.5, gx + gw * 0.5)
    iy_min = jnp.maximum(py - ph * 0.5, gy - gh * 0.5)
    iy_max = jnp.minimum(py + ph * 0.5, gy + gh * 0.5)
    iw = jnp.maximum(ix_max - ix_min, 0.0)
    ih = jnp.maximum(iy_max - iy_min, 0.0)
    inter = iw * ih
    area_a = pw * ph
    area_b = gw * gh
    iou = inter / (area_a + area_b - inter + 1e-10)

    aidx = jnp.argmax(iou, axis=0).astype(jnp.int32)
    a_iota = jax.lax.broadcasted_iota(jnp.int32, (_A, _HW), 0)
    onehot = a_iota == aidx[None, :]
    obj = gconf != 0.0
    objf = obj.astype(jnp.float32)

    mfb = onehot & (gconf >= 1.0)

    sq = ((px - gx) ** 2 + (py - gy) ** 2
          + (pw - gw) ** 2 + (ph - gh) ** 2)
    dconf = jnp.where(onehot & obj, pconf - 1.0, pconf)
    dconf2 = dconf * dconf
    acc_ref[0:_A, :] += jnp.where(mfb, sq, 0.0)
    acc_ref[_A:2 * _A, :] += jnp.where(mfb, dconf2, 0.0)
    acc_ref[2 * _A:3 * _A, :] += dconf2

    @pl.when(state_ref[0] == 0)
    def _resolve_a0():
        any_obj = jnp.max(objf) > 0.0
        j_iota = jax.lax.broadcasted_iota(jnp.int32, (1, _HW), 1)
        big = jnp.int32(2 ** 30)
        j0 = jnp.min(jnp.where(obj, j_iota, big))
        a_here = jnp.sum(jnp.where(j_iota[0] == j0, aidx, 0))
        state_ref[0] = jnp.where(any_obj, 1, 0)
        state_ref[1] = jnp.where(any_obj, a_here, 0)

    a0 = state_ref[1]

    Csel = jax.lax.switch(
        a0, [lambda a=a: pred_ref[0, a * 25:a * 25 + _C, :] for a in range(_A)])
    cmax = jnp.max(Csel, axis=0, keepdims=True)
    ez = jnp.exp(Csel - cmax)
    lz = jnp.log(jnp.sum(ez, axis=0, keepdims=True)) + cmax
    label = jnp.argmax(gcls, axis=0).astype(jnp.int32)
    k_iota = jax.lax.broadcasted_iota(jnp.int32, (_C, _HW), 0)
    picked = jnp.sum(jnp.where(k_iota == label[None, :], Csel, 0.0),
                     axis=0, keepdims=True)
    acc_ref[3 * _A:3 * _A + 1, :] += objf * (lz - picked)


def _loss_step(*refs):
    pred_refs = refs[0:_BPG]
    tgt_refs = refs[_BPG:2 * _BPG]
    sums_ref, state_ref, acc_ref = refs[2 * _BPG], refs[2 * _BPG + 1], refs[2 * _BPG + 2]
    g = pl.program_id(0)

    @pl.when(g == 0)
    def _init():
        acc_ref[...] = jnp.zeros_like(acc_ref)
        state_ref[0] = 0
        state_ref[1] = 0

    for i in range(_BPG):
        _one_image(pred_refs[i], tgt_refs[i], acc_ref, state_ref)

    @pl.when(g == pl.num_programs(0) - 1)
    def _finalize():
        conf_total = jnp.sum(acc_ref[_A:2 * _A, :])
        sums_ref[0] = jnp.sum(acc_ref[0:_A, :])
        sums_ref[1] = conf_total
        sums_ref[2] = jnp.sum(acc_ref[2 * _A:3 * _A, :]) - conf_total
        sums_ref[3] = jnp.sum(acc_ref[3 * _A:3 * _A + 1, :])


def kernel(prediction, target):
    bsize = prediction.shape[0]
    pred = prediction.reshape(bsize, _A * 25, _HW)
    tgt = jnp.transpose(target, (0, 2, 1))

    def _pred_spec(i):
        return pl.BlockSpec((1, _A * 25, _HW), lambda g, i=i: (_BPG * g + i, 0, 0))

    def _tgt_spec(i):
        return pl.BlockSpec((1, 25, _HW), lambda g, i=i: (_BPG * g + i, 0, 0))

    sums, _state = pl.pallas_call(
        _loss_step,
        grid=(bsize // _BPG,),
        in_specs=[_pred_spec(i) for i in range(_BPG)]
                 + [_tgt_spec(i) for i in range(_BPG)],
        out_specs=[
            pl.BlockSpec(memory_space=pltpu.SMEM),
            pl.BlockSpec(memory_space=pltpu.SMEM),
        ],
        out_shape=[
            jax.ShapeDtypeStruct((4,), jnp.float32),
            jax.ShapeDtypeStruct((2,), jnp.int32),
        ],
        scratch_shapes=[
            pltpu.VMEM((3 * _A + 1, _HW), jnp.float32),
        ],
        compiler_params=pltpu.CompilerParams(
            dimension_semantics=("arbitrary",),
        ),
    )(*([pred] * _BPG + [tgt] * _BPG))

    inv_b = 1.0 / bsize
    box_loss = sums[0] * (_LAMBDA_COORD * inv_b)
    conf_loss = sums[1] * inv_b
    noobj_loss = sums[2] * (_LAMBDA_NOOBJ * inv_b)
    cls_loss = sums[3] * inv_b
    return (box_loss, conf_loss, noobj_loss, cls_loss)
